# Initial kernel scaffold; baseline (speedup 1.0000x reference)
import jax
import jax.numpy as jnp
from jax.experimental import pallas as pl


def kernel(node_features, edge_features, memory, source_nodes, timestamps, neighbors, edge_idxs, edge_times, basis_freq, basis_phase, W1, b1, W2, b2):
    raise NotImplementedError("write your pallas kernel here")


# speedup vs baseline: 3.2885x; 3.2885x over previous
import functools

import jax
import jax.numpy as jnp
from jax import lax
from jax.experimental import pallas as pl
from jax.experimental.pallas import tpu as pltpu
from jax.experimental.pallas import tpu_sc as plsc

N_NODES = 10000
N_EDGES = 320000
B = 10000
K = 20
D = 128
DT = 128
DE = 16

NC = 2
NS = 16
L = 16
NW = NC * NS
BP = 10240
RPW = BP // NW
CHUNK = 16
NCHUNK = RPW // CHUNK
IDXC = CHUNK * K
BR = 400
GRID_N = N_NODES // BR
GRID_B = B // BR


def _split_idx(n):
    spans = []
    off = 0
    while off < n:
        w = min(128, n - off)
        spans.append((off, w))
        off += w
    return spans


def _nm_body(a_ref, b_ref, o_ref):
    o_ref[...] = a_ref[...] + b_ref[...]


_nm_add = pl.pallas_call(
    _nm_body,
    grid=(GRID_N,),
    in_specs=[pl.BlockSpec((BR, D), lambda i: (i, 0)),
              pl.BlockSpec((BR, D), lambda i: (i, 0))],
    out_specs=pl.BlockSpec((BR, D), lambda i: (i, 0)),
    out_shape=jax.ShapeDtypeStruct((N_NODES, D), jnp.float32),
)


def _sc_body(nm_h, ef_h, nb_h, ei_h, src_h, snb_h, sef_h, sfe_h,
             nb_v, ei_v, si_v, rows_v, efr_v, acc_v, ace_v, srows_v,
             sem1, sem2, sem3):
    wid = lax.axis_index("s") * NC + lax.axis_index("c")
    base = pl.multiple_of(wid * RPW, RPW)
    ibase = pl.multiple_of(wid * (RPW * K), RPW * K)

    pltpu.sync_copy(nb_h.at[pl.ds(ibase, RPW * K)], nb_v)
    pltpu.sync_copy(ei_h.at[pl.ds(ibase, RPW * K)], ei_v)
    pltpu.sync_copy(src_h.at[pl.ds(base, RPW)], si_v)

    def remap(v, c):
        o = pl.multiple_of(v * L, L)
        nb = nb_v[pl.ds(o, L)]
        e = ei_v[pl.ds(o, L)]
        ei_v[pl.ds(o, L)] = jnp.where(nb == 0, 0, e)
        return c
    lax.fori_loop(0, RPW * K // L, remap, 0)

    cps = [pltpu.async_copy(nm_h.at[si_v.at[pl.ds(o, w)]],
                            srows_v.at[pl.ds(o, w)], sem3)
           for (o, w) in _split_idx(RPW)]
    for cp in cps:
        cp.wait()
    pltpu.sync_copy(srows_v, sfe_h.at[pl.ds(base, RPW)])

    def chunk(ch, c):
        ioff = pl.multiple_of(ch * IDXC, IDXC)
        cpn = [pltpu.async_copy(nm_h.at[nb_v.at[pl.ds(ioff + o, w)]],
                                rows_v.at[pl.ds(o, w)], sem1)
               for (o, w) in _split_idx(IDXC)]
        cpe = [pltpu.async_copy(ef_h.at[ei_v.at[pl.ds(ioff + o, w)]],
                                efr_v.at[pl.ds(o, w)], sem2)
               for (o, w) in _split_idx(IDXC)]
        for cp in cpn + cpe:
            cp.wait()

        def row(r, cc):
            rb = r * K
            for j in range(D // L):
                s = rows_v[rb, pl.ds(j * L, L)]
                for kk in range(1, K):
                    s = s + rows_v[rb + kk, pl.ds(j * L, L)]
                acc_v[r, pl.ds(j * L, L)] = s
            se = efr_v[rb, :]
            for kk in range(1, K):
                se = se + efr_v[rb + kk, :]
            ace_v[r, :] = se
            return cc
        lax.fori_loop(0, CHUNK, row, 0)

        ob = pl.multiple_of(base + ch * CHUNK, CHUNK)
        pltpu.sync_copy(acc_v, snb_h.at[pl.ds(ob, CHUNK)])
        pltpu.sync_copy(ace_v, sef_h.at[pl.ds(ob, CHUNK)])
        return c
    lax.fori_loop(0, NCHUNK, chunk, 0)


def _sc_gather(nm, ef, nb_flat, ei_flat, src):
    mesh = plsc.VectorSubcoreMesh(core_axis_name="c", subcore_axis_name="s")
    out_type = (
        jax.ShapeDtypeStruct((BP, D), jnp.float32),
        jax.ShapeDtypeStruct((BP, DE), jnp.float32),
        jax.ShapeDtypeStruct((BP, D), jnp.float32),
    )
    scratch = [
        pltpu.VMEM((RPW * K,), jnp.int32),
        pltpu.VMEM((RPW * K,), jnp.int32),
        pltpu.VMEM((RPW,), jnp.int32),
        pltpu.VMEM((IDXC, D), jnp.float32),
        pltpu.VMEM((IDXC, DE), jnp.float32),
        pltpu.VMEM((CHUNK, D), jnp.float32),
        pltpu.VMEM((CHUNK, DE), jnp.float32),
        pltpu.VMEM((RPW, D), jnp.float32),
        pltpu.SemaphoreType.DMA,
        pltpu.SemaphoreType.DMA,
        pltpu.SemaphoreType.DMA,
    ]
    k = functools.partial(
        pl.kernel, mesh=mesh, out_type=out_type, scratch_types=scratch,
        compiler_params=pltpu.CompilerParams(use_tc_tiling_on_sc=False),
    )(_sc_body)
    return k(nm, ef, nb_flat, ei_flat, src)


def _fin_body(ts_ref, et_ref, nb_ref, snb_ref, sef_ref, sfe_ref,
              nm0_ref, ef0_ref, f_ref, p_ref,
              w1a_ref, w1b_ref, w1c_ref, b1_ref,
              w2a_ref, w2b_ref, w2c_ref, b2_ref, o_ref):
    ts = ts_ref[...]
    et = et_ref[...]
    nb = nb_ref[...]
    f = f_ref[...]
    p = p_ref[...]
    delta = ts - et
    mf = jnp.where(nb == 0, 0.0, 1.0)
    cnt = jnp.sum(mf, axis=1, keepdims=True)
    cnt0 = K - cnt
    acc = jnp.cos(delta[:, 0:1] * f + p) * mf[:, 0:1]
    for kk in range(1, K):
        acc = acc + jnp.cos(delta[:, kk:kk + 1] * f + p) * mf[:, kk:kk + 1]
    snb = snb_ref[...] - cnt0 * nm0_ref[...]
    sef = sef_ref[...] - cnt0 * ef0_ref[...]
    pre = (jnp.dot(snb, w1a_ref[...], preferred_element_type=jnp.float32)
           + jnp.dot(acc, w1b_ref[...], preferred_element_type=jnp.float32)
           + jnp.dot(sef, w1c_ref[...], preferred_element_type=jnp.float32)
           + cnt * b1_ref[...])
    ns = jnp.maximum(pre, 0.0)
    c0 = jnp.dot(jnp.cos(p), w2c_ref[...],
                 preferred_element_type=jnp.float32) + b2_ref[...]
    o_ref[...] = (jnp.dot(ns, w2a_ref[...], preferred_element_type=jnp.float32)
                  + jnp.dot(sfe_ref[...], w2b_ref[...],
                            preferred_element_type=jnp.float32)
                  + c0)


def _const2(shape):
    return pl.BlockSpec(shape, lambda i: (0, 0))


_finish_in_specs = [
    pl.BlockSpec((BR, 1), lambda i: (i, 0)),
    pl.BlockSpec((BR, K), lambda i: (i, 0)),
    pl.BlockSpec((BR, K), lambda i: (i, 0)),
    pl.BlockSpec((BR, D), lambda i: (i, 0)),
    pl.BlockSpec((BR, DE), lambda i: (i, 0)),
    pl.BlockSpec((BR, D), lambda i: (i, 0)),
    _const2((1, D)),
    _const2((1, DE)),
    _const2((1, DT)),
    _const2((1, DT)),
    _const2((D, D)),
    _const2((DT, D)),
    _const2((DE, D)),
    _const2((1, D)),
    _const2((D, D)),
    _const2((D, D)),
    _const2((DT, D)),
    _const2((1, D)),
]
_finish_out_spec = pl.BlockSpec((BR, D), lambda i: (i, 0))

_finish = pl.pallas_call(
    _fin_body,
    grid=(GRID_B,),
    in_specs=_finish_in_specs,
    out_specs=_finish_out_spec,
    out_shape=jax.ShapeDtypeStruct((B, D), jnp.float32),
)


def kernel(node_features, edge_features, memory, source_nodes, timestamps,
           neighbors, edge_idxs, edge_times, basis_freq, basis_phase,
           W1, b1, W2, b2):
    nm = _nm_add(node_features, memory)

    pad = BP - B
    nb_p = jnp.pad(neighbors.astype(jnp.int32), ((0, pad), (0, 0)))
    ei_p = jnp.pad(edge_idxs.astype(jnp.int32), ((0, pad), (0, 0)))
    src_p = jnp.pad(source_nodes.astype(jnp.int32), (0, pad))

    snb, sef, sfe = _sc_gather(nm, edge_features,
                               nb_p.reshape(-1), ei_p.reshape(-1), src_p)

    out = _finish(
        timestamps.reshape(B, 1), edge_times, neighbors.astype(jnp.int32),
        snb[:B], sef[:B], sfe[:B],
        nm[0:1], edge_features[0:1],
        basis_freq.reshape(1, DT), basis_phase.reshape(1, DT),
        W1[:D], W1[D:D + DT], W1[D + DT:], b1.reshape(1, D),
        W2[:D], W2[D:2 * D], W2[2 * D:], b2.reshape(1, D),
    )
    return out


# speedup vs baseline: 5.2551x; 1.5980x over previous
import functools

import jax
import jax.numpy as jnp
from jax import lax
from jax.experimental import pallas as pl
from jax.experimental.pallas import tpu as pltpu
from jax.experimental.pallas import tpu_sc as plsc

N_NODES = 10000
N_EDGES = 320000
B = 10000
K = 20
D = 128
DT = 128
DE = 16

NC = 2
NS = 16
L = 16
NW = NC * NS
BP = 10240
RPW = BP // NW
CHUNK = 16
NCHUNK = RPW // CHUNK
IDXC = CHUNK * K
BR = 400
GRID_N = N_NODES // BR
GRID_B = B // BR


def _split_idx(n):
    spans = []
    off = 0
    while off < n:
        w = min(128, n - off)
        spans.append((off, w))
        off += w
    return spans


def _nm_body(a_ref, b_ref, o_ref):
    o_ref[...] = a_ref[...] + b_ref[...]


_nm_add = pl.pallas_call(
    _nm_body,
    grid=(GRID_N,),
    in_specs=[pl.BlockSpec((BR, D), lambda i: (i, 0)),
              pl.BlockSpec((BR, D), lambda i: (i, 0))],
    out_specs=pl.BlockSpec((BR, D), lambda i: (i, 0)),
    out_shape=jax.ShapeDtypeStruct((N_NODES, D), jnp.float32),
)


def _sc_body(nm_h, ef_h, nb_h, ei_h, src_h, snb_h, sef_h, sfe_h,
             nb_v, ei_v, si_v, rows_v, efr_v, acc_v, ace_v,
             semn0, semn1, seme0, seme1, sems):
    wid = lax.axis_index("s") * NC + lax.axis_index("c")
    base = pl.multiple_of(wid * RPW, RPW)
    ibase = pl.multiple_of(wid * (RPW * K), RPW * K)
    semn = [semn0, semn1]
    seme = [seme0, seme1]

    pltpu.sync_copy(nb_h.at[pl.ds(ibase, RPW * K)], nb_v)
    pltpu.sync_copy(ei_h.at[pl.ds(ibase, RPW * K)], ei_v)
    pltpu.sync_copy(src_h.at[pl.ds(base, RPW)], si_v)

    def remap(v, c):
        o = pl.multiple_of(v * L, L)
        nb = nb_v[pl.ds(o, L)]
        e = ei_v[pl.ds(o, L)]
        ei_v[pl.ds(o, L)] = jnp.where(nb == 0, 0, e)
        return c
    lax.fori_loop(0, RPW * K // L, remap, 0)

    src_cps = [pltpu.make_async_copy(nm_h.at[si_v.at[pl.ds(o, w)]],
                                     rows_v.at[0].at[pl.ds(o, w)], sems)
               for (o, w) in _split_idx(RPW)]
    for cp in src_cps:
        cp.start()
    for cp in src_cps:
        cp.wait()
    pltpu.sync_copy(rows_v.at[0], sfe_h.at[pl.ds(base, RPW)])

    def dmas(ch, b):
        ioff = pl.multiple_of(ch * IDXC, IDXC)
        cps = []
        for (o, w) in _split_idx(IDXC):
            cps.append(pltpu.make_async_copy(
                nm_h.at[nb_v.at[pl.ds(ioff + o, w)]],
                rows_v.at[b].at[pl.ds(o, w)], semn[b]))
            cps.append(pltpu.make_async_copy(
                ef_h.at[ei_v.at[pl.ds(ioff + o, w)]],
                efr_v.at[b].at[pl.ds(o, w)], seme[b]))
        return cps

    def issue(ch, b):
        for cp in dmas(ch, b):
            cp.start()

    def wait(ch, b):
        for cp in dmas(ch, b):
            cp.wait()

    def accum(ch, b):
        def row(r, cc):
            rb = r * K
            for j in range(D // L):
                s = rows_v[b, rb, pl.ds(j * L, L)]
                for kk in range(1, K):
                    s = s + rows_v[b, rb + kk, pl.ds(j * L, L)]
                acc_v[r, pl.ds(j * L, L)] = s
            se = efr_v[b, rb, :]
            for kk in range(1, K):
                se = se + efr_v[b, rb + kk, :]
            ace_v[r, :] = se
            return cc
        lax.fori_loop(0, CHUNK, row, 0)
        ob = pl.multiple_of(base + ch * CHUNK, CHUNK)
        pltpu.sync_copy(acc_v, snb_h.at[pl.ds(ob, CHUNK)])
        pltpu.sync_copy(ace_v, sef_h.at[pl.ds(ob, CHUNK)])

    issue(0, 0)

    def outer(i, c):
        ch0 = i * 2
        ch1 = ch0 + 1
        issue(ch1, 1)
        wait(ch0, 0)
        accum(ch0, 0)

        @pl.when(ch1 + 1 < NCHUNK)
        def _():
            issue(ch1 + 1, 0)
        wait(ch1, 1)
        accum(ch1, 1)
        return c
    lax.fori_loop(0, NCHUNK // 2, outer, 0)


def _sc_gather(nm, ef, nb_flat, ei_flat, src):
    mesh = plsc.VectorSubcoreMesh(core_axis_name="c", subcore_axis_name="s")
    out_type = (
        jax.ShapeDtypeStruct((BP, D), jnp.float32),
        jax.ShapeDtypeStruct((BP, DE), jnp.float32),
        jax.ShapeDtypeStruct((BP, D), jnp.float32),
    )
    scratch = [
        pltpu.VMEM((RPW * K,), jnp.int32),
        pltpu.VMEM((RPW * K,), jnp.int32),
        pltpu.VMEM((RPW,), jnp.int32),
        pltpu.VMEM((2, IDXC, D), jnp.float32),
        pltpu.VMEM((2, IDXC, DE), jnp.float32),
        pltpu.VMEM((CHUNK, D), jnp.float32),
        pltpu.VMEM((CHUNK, DE), jnp.float32),
        pltpu.SemaphoreType.DMA,
        pltpu.SemaphoreType.DMA,
        pltpu.SemaphoreType.DMA,
        pltpu.SemaphoreType.DMA,
        pltpu.SemaphoreType.DMA,
    ]
    k = functools.partial(
        pl.kernel, mesh=mesh, out_type=out_type, scratch_types=scratch,
        compiler_params=pltpu.CompilerParams(use_tc_tiling_on_sc=False),
    )(_sc_body)
    return k(nm, ef, nb_flat, ei_flat, src)


def _f1_body(ts_ref, et_ref, nb_ref, f_ref, p_ref, o_ref):
    delta = ts_ref[...] - et_ref[...]
    mf = jnp.where(nb_ref[...] == 0, 0.0, 1.0)
    f = f_ref[...]
    p = p_ref[...]
    acc = jnp.cos(delta[:, 0:1] * f + p) * mf[:, 0:1]
    for kk in range(1, K):
        acc = acc + jnp.cos(delta[:, kk:kk + 1] * f + p) * mf[:, kk:kk + 1]
    o_ref[...] = acc


_f1_in_specs = [
    pl.BlockSpec((BR, 1), lambda i: (i, 0)),
    pl.BlockSpec((BR, K), lambda i: (i, 0)),
    pl.BlockSpec((BR, K), lambda i: (i, 0)),
    pl.BlockSpec((1, DT), lambda i: (0, 0)),
    pl.BlockSpec((1, DT), lambda i: (0, 0)),
]
_f1_out_spec = pl.BlockSpec((BR, DT), lambda i: (i, 0))

_f1 = pl.pallas_call(
    _f1_body,
    grid=(GRID_B,),
    in_specs=_f1_in_specs,
    out_specs=_f1_out_spec,
    out_shape=jax.ShapeDtypeStruct((B, DT), jnp.float32),
)


def _f2_body(nb_ref, snb_ref, sef_ref, sfe_ref, tte_ref, nm0_ref, ef0_ref,
             p_ref, w1a_ref, w1b_ref, w1c_ref, b1_ref,
             w2a_ref, w2b_ref, w2c_ref, b2_ref, o_ref):
    mf = jnp.where(nb_ref[...] == 0, 0.0, 1.0)
    cnt = jnp.sum(mf, axis=1, keepdims=True)
    cnt0 = K - cnt
    snb = snb_ref[...] - cnt0 * nm0_ref[...]
    sef = sef_ref[...] - cnt0 * ef0_ref[...]
    pre = (jnp.dot(snb, w1a_ref[...], preferred_element_type=jnp.float32)
           + jnp.dot(tte_ref[...], w1b_ref[...],
                     preferred_element_type=jnp.float32)
           + jnp.dot(sef, w1c_ref[...], preferred_element_type=jnp.float32)
           + cnt * b1_ref[...])
    ns = jnp.maximum(pre, 0.0)
    c0 = jnp.dot(jnp.cos(p_ref[...]), w2c_ref[...],
                 preferred_element_type=jnp.float32) + b2_ref[...]
    o_ref[...] = (jnp.dot(ns, w2a_ref[...], preferred_element_type=jnp.float32)
                  + jnp.dot(sfe_ref[...], w2b_ref[...],
                            preferred_element_type=jnp.float32)
                  + c0)


def _const2(shape):
    return pl.BlockSpec(shape, lambda i: (0, 0))


_f2_in_specs = [
    pl.BlockSpec((BR, K), lambda i: (i, 0)),
    pl.BlockSpec((BR, D), lambda i: (i, 0)),
    pl.BlockSpec((BR, DE), lambda i: (i, 0)),
    pl.BlockSpec((BR, D), lambda i: (i, 0)),
    pl.BlockSpec((BR, DT), lambda i: (i, 0)),
    _const2((1, D)),
    _const2((1, DE)),
    _const2((1, DT)),
    _const2((D, D)),
    _const2((DT, D)),
    _const2((DE, D)),
    _const2((1, D)),
    _const2((D, D)),
    _const2((D, D)),
    _const2((DT, D)),
    _const2((1, D)),
]
_f2_out_spec = pl.BlockSpec((BR, D), lambda i: (i, 0))

_f2 = pl.pallas_call(
    _f2_body,
    grid=(GRID_B,),
    in_specs=_f2_in_specs,
    out_specs=_f2_out_spec,
    out_shape=jax.ShapeDtypeStruct((B, D), jnp.float32),
)


def kernel(node_features, edge_features, memory, source_nodes, timestamps,
           neighbors, edge_idxs, edge_times, basis_freq, basis_phase,
           W1, b1, W2, b2):
    nm = _nm_add(node_features, memory)

    pad = BP - B
    nb32 = neighbors.astype(jnp.int32)
    nb_p = jnp.pad(nb32, ((0, pad), (0, 0)))
    ei_p = jnp.pad(edge_idxs.astype(jnp.int32), ((0, pad), (0, 0)))
    src_p = jnp.pad(source_nodes.astype(jnp.int32), (0, pad))

    snb, sef, sfe = _sc_gather(nm, edge_features,
                               nb_p.reshape(-1), ei_p.reshape(-1), src_p)

    tte = _f1(timestamps.reshape(B, 1), edge_times, nb32,
              basis_freq.reshape(1, DT), basis_phase.reshape(1, DT))

    out = _f2(
        nb32, snb[:B], sef[:B], sfe[:B], tte,
        nm[0:1], edge_features[0:1], basis_phase.reshape(1, DT),
        W1[:D], W1[D:D + DT], W1[D + DT:], b1.reshape(1, D),
        W2[:D], W2[D:2 * D], W2[2 * D:], b2.reshape(1, D),
    )
    return out
